# baseline (speedup 1.0000x reference)
import functools

import jax
import jax.numpy as jnp
from jax import lax
from jax.experimental import pallas as pl
from jax.experimental.pallas import tpu as pltpu
from jax.experimental.pallas import tpu_sc as plsc

E = 50000
E_NB = 400000
D_EDGE = 512
D_SBF = 64
D_QUAD = 64

HALF = 25088
E_PAD = 2 * HALF
QUARTER = HALF // 2
TRASH = QUARTER
ACC_ROWS = QUARTER + 8
PER_TILE = HALF // 16
PER_TILE_Q = QUARTER // 16

CHUNK = 512
PCH = 256
MAIN_PER_W = 24576
N_MAIN = MAIN_PER_W // PCH
EXTRA_BASE = 16 * MAIN_PER_W
TAIL2_BASE = EXTRA_BASE + 13 * CHUNK

GB = 112
N_GB = PER_TILE // GB

BLK = 1024
GRID = E_PAD // BLK
INV_SQRT2 = 0.7071067811865476

@functools.cache
def _mesh():
    return plsc.VectorSubcoreMesh(core_axis_name="c", subcore_axis_name="s")


def _segsum_body(sbf_hbm, idx_hbm, zeros_hbm, out_hbm, rows_v, idx_sc,
                 idx_all, acc_sh, sem, sem2):
    c = lax.axis_index("c")
    s = lax.axis_index("s")
    mb = s * MAIN_PER_W

    pltpu.sync_copy(idx_hbm.at[pl.ds(mb, MAIN_PER_W)],
                    idx_all.at[pl.ds(0, MAIN_PER_W)])

    @pl.when(s < 13)
    def _():
        pltpu.sync_copy(idx_hbm.at[pl.ds(EXTRA_BASE + s * CHUNK, CHUNK)],
                        idx_all.at[pl.ds(MAIN_PER_W, CHUNK)])

    @pl.when(s == 13)
    def _():
        pltpu.sync_copy(idx_hbm.at[pl.ds(TAIL2_BASE, 128)],
                        idx_all.at[pl.ds(MAIN_PER_W, 128)])

    for p in range(2):
        lo = c * HALF + p * QUARTER
        hi = lo + QUARTER

        pltpu.sync_copy(zeros_hbm, acc_sh.at[pl.ds(s * PER_TILE_Q, PER_TILE_Q)])

        @pl.when(s == 0)
        def _():
            pltpu.sync_copy(zeros_hbm.at[pl.ds(0, 8)],
                            acc_sh.at[pl.ds(QUARTER, 8)])

        plsc.subcore_barrier()

        n_lo = jnp.int32(-1)
        n_hi = jnp.int32(-2)
        for n in range(N_MAIN):
            f = idx_all[pl.ds(n * PCH, 16)][0]
            l = idx_all[pl.ds(n * PCH + PCH - 16, 16)][15]
            inr = (f < hi) & (l >= lo)
            n_lo = jnp.where(inr & (n_lo < 0), n, n_lo)
            n_hi = jnp.where(inr, n, n_hi)

        def transform(base, nvec, slot, lo=lo, hi=hi):
            for v in range(nvec):
                vec = idx_all[pl.ds(base + v * 16, 16)]
                inr = (vec >= lo) & (vec < hi)
                idx_sc[slot * 2 + v // 8, pl.ds((v % 8) * 16, 16)] = jnp.where(
                    inr, vec - lo, TRASH)

        def issue(k):
            pltpu.async_copy(sbf_hbm.at[pl.ds(mb + k * PCH, PCH)],
                             rows_v.at[pl.ds((k % 3) * PCH, PCH)], sem)

        def drain_scatter():
            for _ in range(PCH // 128):
                pltpu.make_async_copy(sbf_hbm.at[pl.ds(0, 128)],
                                      rows_v.at[pl.ds(0, 128)], sem2).wait()

        @pl.when(n_hi >= n_lo)
        def _():
            issue(n_lo)

            @pl.when(n_lo + 1 <= n_hi)
            def _():
                issue(n_lo + 1)

            def body(n, carry):
                @pl.when(n >= n_lo + 2)
                def _():
                    drain_scatter()

                @pl.when(n + 2 <= n_hi)
                def _():
                    issue(n + 2)

                transform(n * PCH, PCH // 16, n % 3)
                pltpu.make_async_copy(
                    sbf_hbm.at[pl.ds(mb + n * PCH, PCH)],
                    rows_v.at[pl.ds((n % 3) * PCH, PCH)], sem).wait()
                for j in range(PCH // 128):
                    pltpu.async_copy(
                        rows_v.at[pl.ds((n % 3) * PCH + j * 128, 128)],
                        acc_sh.at[idx_sc.at[(n % 3) * 2 + j]], sem2, add=True)
                return carry

            lax.fori_loop(n_lo, n_hi + 1, body, 0)
            drain_scatter()

            @pl.when(n_hi > n_lo)
            def _():
                drain_scatter()

        def extra(base, nrows, lo=lo, hi=hi):
            nsub = nrows // 128
            first = idx_all[pl.ds(MAIN_PER_W, 16)][0]
            last = idx_all[pl.ds(MAIN_PER_W + nrows - 16, 16)][15]

            @pl.when((first < hi) & (last >= lo))
            def _():
                transform(MAIN_PER_W, nrows // 16, 0)
                pltpu.sync_copy(sbf_hbm.at[pl.ds(base, nrows)],
                                rows_v.at[pl.ds(0, nrows)])
                for j in range(nsub):
                    pltpu.sync_copy(rows_v.at[pl.ds(j * 128, 128)],
                                    acc_sh.at[idx_sc.at[j]], add=True)

        @pl.when(s < 13)
        def _():
            extra(EXTRA_BASE + s * CHUNK, CHUNK)

        @pl.when(s == 13)
        def _():
            extra(TAIL2_BASE, 128)

        plsc.subcore_barrier()
        pltpu.sync_copy(acc_sh.at[pl.ds(s * PER_TILE_Q, PER_TILE_Q)],
                        out_hbm.at[pl.ds(lo + s * PER_TILE_Q, PER_TILE_Q),
                                   pl.ds(0, D_SBF)])
        pltpu.sync_copy(acc_sh.at[pl.ds(s * PER_TILE_Q, PER_TILE_Q)],
                        out_hbm.at[pl.ds(lo + s * PER_TILE_Q, PER_TILE_Q),
                                   pl.ds(D_SBF, D_SBF)])
        plsc.subcore_barrier()


@functools.cache
def _segsum():
    return pl.kernel(
        _segsum_body,
        mesh=_mesh(),
        out_type=jax.ShapeDtypeStruct((E_PAD, 2 * D_SBF), jnp.float32),
        scratch_types=[
            pltpu.VMEM((3 * PCH, D_SBF), jnp.float32),
            pltpu.VMEM((6, 128), jnp.int32),
            pltpu.VMEM((MAIN_PER_W + CHUNK,), jnp.int32),
            pltpu.VMEM_SHARED((ACC_ROWS, D_SBF), jnp.float32),
            pltpu.SemaphoreType.DMA,
            pltpu.SemaphoreType.DMA,
        ],
        compiler_params=pltpu.CompilerParams(use_tc_tiling_on_sc=False),
    )


def _gather_body(x_hbm, idxs_hbm, out_hbm, idx_v, rows_v, sem):
    c = lax.axis_index("c")
    s = lax.axis_index("s")
    wid = s * 2 + c
    base = wid * PER_TILE
    sbases = [jnp.minimum(base + j * GB, E - GB) for j in range(N_GB)]
    for j in range(N_GB):
        pltpu.sync_copy(idxs_hbm.at[pl.ds(sbases[j], GB)], idx_v.at[j])
    for half in range(2):
        descs = []
        for j in range(N_GB // 2):
            g = half * (N_GB // 2) + j
            descs.append(pltpu.async_copy(x_hbm.at[idx_v.at[g]],
                                          rows_v.at[pl.ds(j * GB, GB)], sem))
        for d in descs:
            d.wait()
        for j in range(N_GB // 2):
            g = half * (N_GB // 2) + j
            pltpu.sync_copy(rows_v.at[pl.ds(j * GB, GB)],
                            out_hbm.at[pl.ds(sbases[g], GB)])


@functools.cache
def _gather():
    return pl.kernel(
        _gather_body,
        mesh=_mesh(),
        out_type=jax.ShapeDtypeStruct((E_PAD, 2 * D_QUAD), jnp.float32),
        scratch_types=[
            pltpu.VMEM((N_GB, GB), jnp.int32),
            pltpu.VMEM((PER_TILE // 2, 2 * D_QUAD), jnp.float32),
            pltpu.SemaphoreType.DMA,
        ],
    )


def _prep_body(wb_ref, wst_ref, wts_ref, wbt_ref, fold_ref, wstp_ref,
               wtsp_ref):
    z = jnp.zeros((D_SBF, D_QUAD * D_QUAD), jnp.float32)
    w3 = wb_ref[...]
    wbt = jnp.transpose(w3, (0, 2, 1)).reshape(D_SBF, D_QUAD * D_QUAD)
    wbt_ref[...] = jnp.concatenate([wbt, z], axis=0)
    r = lax.broadcasted_iota(jnp.int32, (D_QUAD * D_QUAD, 2 * D_QUAD), 0)
    c = lax.broadcasted_iota(jnp.int32, (D_QUAD * D_QUAD, 2 * D_QUAD), 1)
    fold_ref[...] = jnp.where(r // D_QUAD == c, 1.0, 0.0)
    zw = jnp.zeros((D_QUAD, D_EDGE), jnp.float32)
    wstp_ref[...] = jnp.concatenate([wst_ref[...], zw], axis=0)
    wtsp_ref[...] = jnp.concatenate([wts_ref[...], zw], axis=0)


def _h_body(m_ref, wd_ref, h_ref):
    h = jnp.dot(m_ref[...], wd_ref[...], preferred_element_type=jnp.float32)
    h_ref[...] = h * jax.nn.sigmoid(h)


def _bilinear_body(h_ref, s_ref, wbt_ref, fold_ref, x_ref):
    h = h_ref[...]
    t = jnp.dot(s_ref[...], wbt_ref[...], preferred_element_type=jnp.float32)
    hrep = pltpu.repeat(h, D_QUAD, axis=1)
    x_ref[...] = jnp.dot(t * hrep, fold_ref[...],
                         preferred_element_type=jnp.float32)


def _up_body(x_ref, xs_ref, ws_ref, wt_ref, o_ref):
    a = jnp.dot(x_ref[...], ws_ref[...], preferred_element_type=jnp.float32)
    b = jnp.dot(xs_ref[...], wt_ref[...], preferred_element_type=jnp.float32)
    o_ref[...] = (a * jax.nn.sigmoid(a) + b * jax.nn.sigmoid(b)) * INV_SQRT2


def kernel(m_st, sbf, idx_swap, edge_nb_idx, edge_nb_ragged_idx,
           W_down, W_bilinear, W_up_st, W_up_ts):
    del edge_nb_ragged_idx
    wbt, fold, wstp, wtsp = pl.pallas_call(
        _prep_body,
        out_shape=(
            jax.ShapeDtypeStruct((2 * D_SBF, D_QUAD * D_QUAD), jnp.float32),
            jax.ShapeDtypeStruct((D_QUAD * D_QUAD, 2 * D_QUAD), jnp.float32),
            jax.ShapeDtypeStruct((2 * D_QUAD, D_EDGE), jnp.float32),
            jax.ShapeDtypeStruct((2 * D_QUAD, D_EDGE), jnp.float32),
        ),
    )(W_bilinear, W_up_st, W_up_ts)
    zeros = jnp.zeros((PER_TILE_Q, D_SBF), jnp.float32)

    h_pad = pl.pallas_call(
        _h_body,
        grid=(GRID,),
        in_specs=[
            pl.BlockSpec((BLK, D_EDGE), lambda i: (i, 0)),
            pl.BlockSpec((D_EDGE, D_QUAD), lambda i: (0, 0)),
        ],
        out_specs=pl.BlockSpec((BLK, D_QUAD), lambda i: (i, 0)),
        out_shape=jax.ShapeDtypeStruct((E_PAD, D_QUAD), jnp.float32),
    )(m_st, W_down)

    s_pad = _segsum()(sbf, edge_nb_idx.astype(jnp.int32), zeros)

    x_pad = pl.pallas_call(
        _bilinear_body,
        grid=(GRID,),
        in_specs=[
            pl.BlockSpec((BLK, D_QUAD), lambda i: (i, 0)),
            pl.BlockSpec((BLK, 2 * D_SBF), lambda i: (i, 0)),
            pl.BlockSpec((2 * D_SBF, D_QUAD * D_QUAD), lambda i: (0, 0)),
            pl.BlockSpec((D_QUAD * D_QUAD, 2 * D_QUAD), lambda i: (0, 0)),
        ],
        out_specs=pl.BlockSpec((BLK, 2 * D_QUAD), lambda i: (i, 0)),
        out_shape=jax.ShapeDtypeStruct((E_PAD, 2 * D_QUAD), jnp.float32),
    )(h_pad, s_pad, wbt, fold)

    x_sw = _gather()(x_pad, idx_swap.astype(jnp.int32))

    out = pl.pallas_call(
        _up_body,
        grid=(GRID,),
        in_specs=[
            pl.BlockSpec((BLK, 2 * D_QUAD), lambda i: (i, 0)),
            pl.BlockSpec((BLK, 2 * D_QUAD), lambda i: (i, 0)),
            pl.BlockSpec((2 * D_QUAD, D_EDGE), lambda i: (0, 0)),
            pl.BlockSpec((2 * D_QUAD, D_EDGE), lambda i: (0, 0)),
        ],
        out_specs=pl.BlockSpec((BLK, D_EDGE), lambda i: (i, 0)),
        out_shape=jax.ShapeDtypeStruct((E, D_EDGE), jnp.float32),
    )(x_pad, x_sw, wstp, wtsp)

    return out


# speedup vs baseline: 1.0001x; 1.0001x over previous
import functools

import jax
import jax.numpy as jnp
from jax import lax
from jax.experimental import pallas as pl
from jax.experimental.pallas import tpu as pltpu
from jax.experimental.pallas import tpu_sc as plsc

E = 50000
E_NB = 400000
D_EDGE = 512
D_SBF = 64
D_QUAD = 64

HALF = 25088
E_PAD = 2 * HALF
QUARTER = HALF // 2
TRASH = QUARTER
ACC_ROWS = QUARTER + 8
PER_TILE = HALF // 16
PER_TILE_Q = QUARTER // 16

CHUNK = 512
PCH = 256
MAIN_PER_W = 24576
N_MAIN = MAIN_PER_W // PCH
EXTRA_BASE = 16 * MAIN_PER_W
TAIL2_BASE = EXTRA_BASE + 13 * CHUNK

GB = 112
N_GB = PER_TILE // GB

BLK = 1024
GRID = E_PAD // BLK
INV_SQRT2 = 0.7071067811865476

@functools.cache
def _mesh():
    return plsc.VectorSubcoreMesh(core_axis_name="c", subcore_axis_name="s")


def _segsum_body(sbf_hbm, idx_hbm, zeros_hbm, out_hbm, rows_v, idx_sc,
                 idx_all, acc_sh, sem, sem2):
    c = lax.axis_index("c")
    s = lax.axis_index("s")
    mb = s * MAIN_PER_W

    pltpu.sync_copy(idx_hbm.at[pl.ds(mb, MAIN_PER_W)],
                    idx_all.at[pl.ds(0, MAIN_PER_W)])

    @pl.when(s < 13)
    def _():
        pltpu.sync_copy(idx_hbm.at[pl.ds(EXTRA_BASE + s * CHUNK, CHUNK)],
                        idx_all.at[pl.ds(MAIN_PER_W, CHUNK)])

    @pl.when(s == 13)
    def _():
        pltpu.sync_copy(idx_hbm.at[pl.ds(TAIL2_BASE, 128)],
                        idx_all.at[pl.ds(MAIN_PER_W, 128)])

    for p in range(2):
        lo = c * HALF + p * QUARTER
        hi = lo + QUARTER

        pltpu.sync_copy(zeros_hbm, acc_sh.at[pl.ds(s * PER_TILE_Q, PER_TILE_Q)])

        @pl.when(s == 0)
        def _():
            pltpu.sync_copy(zeros_hbm.at[pl.ds(0, 8)],
                            acc_sh.at[pl.ds(QUARTER, 8)])

        plsc.subcore_barrier()

        n_lo = jnp.int32(-1)
        n_hi = jnp.int32(-2)
        for n in range(N_MAIN):
            f = idx_all[pl.ds(n * PCH, 16)][0]
            l = idx_all[pl.ds(n * PCH + PCH - 16, 16)][15]
            inr = (f < hi) & (l >= lo)
            n_lo = jnp.where(inr & (n_lo < 0), n, n_lo)
            n_hi = jnp.where(inr, n, n_hi)

        def transform(base, nvec, slot, lo=lo, hi=hi):
            for v in range(nvec):
                vec = idx_all[pl.ds(base + v * 16, 16)]
                inr = (vec >= lo) & (vec < hi)
                idx_sc[slot * 2 + v // 8, pl.ds((v % 8) * 16, 16)] = jnp.where(
                    inr, vec - lo, TRASH)

        def issue(k):
            pltpu.async_copy(sbf_hbm.at[pl.ds(mb + k * PCH, PCH)],
                             rows_v.at[pl.ds((k % 3) * PCH, PCH)], sem)

        def drain_scatter():
            for _ in range(PCH // 128):
                pltpu.make_async_copy(sbf_hbm.at[pl.ds(0, 128)],
                                      rows_v.at[pl.ds(0, 128)], sem2).wait()

        @pl.when(n_hi >= n_lo)
        def _():
            issue(n_lo)

            @pl.when(n_lo + 1 <= n_hi)
            def _():
                issue(n_lo + 1)

            def body(n, carry):
                @pl.when(n >= n_lo + 2)
                def _():
                    drain_scatter()

                @pl.when(n + 2 <= n_hi)
                def _():
                    issue(n + 2)

                transform(n * PCH, PCH // 16, n % 3)
                pltpu.make_async_copy(
                    sbf_hbm.at[pl.ds(mb + n * PCH, PCH)],
                    rows_v.at[pl.ds((n % 3) * PCH, PCH)], sem).wait()
                for j in range(PCH // 128):
                    pltpu.async_copy(
                        rows_v.at[pl.ds((n % 3) * PCH + j * 128, 128)],
                        acc_sh.at[idx_sc.at[(n % 3) * 2 + j]], sem2, add=True)
                return carry

            lax.fori_loop(n_lo, n_hi + 1, body, 0)
            drain_scatter()

            @pl.when(n_hi > n_lo)
            def _():
                drain_scatter()

        def extra(base, nrows, lo=lo, hi=hi):
            nsub = nrows // 128
            first = idx_all[pl.ds(MAIN_PER_W, 16)][0]
            last = idx_all[pl.ds(MAIN_PER_W + nrows - 16, 16)][15]

            @pl.when((first < hi) & (last >= lo))
            def _():
                transform(MAIN_PER_W, nrows // 16, 0)
                pltpu.sync_copy(sbf_hbm.at[pl.ds(base, nrows)],
                                rows_v.at[pl.ds(0, nrows)])
                for j in range(nsub):
                    pltpu.sync_copy(rows_v.at[pl.ds(j * 128, 128)],
                                    acc_sh.at[idx_sc.at[j]], add=True)

        @pl.when(s < 13)
        def _():
            extra(EXTRA_BASE + s * CHUNK, CHUNK)

        @pl.when(s == 13)
        def _():
            extra(TAIL2_BASE, 128)

        plsc.subcore_barrier()
        pltpu.sync_copy(acc_sh.at[pl.ds(s * PER_TILE_Q, PER_TILE_Q)],
                        out_hbm.at[pl.ds(lo + s * PER_TILE_Q, PER_TILE_Q),
                                   pl.ds(0, D_SBF)])
        pltpu.sync_copy(acc_sh.at[pl.ds(s * PER_TILE_Q, PER_TILE_Q)],
                        out_hbm.at[pl.ds(lo + s * PER_TILE_Q, PER_TILE_Q),
                                   pl.ds(D_SBF, D_SBF)])
        plsc.subcore_barrier()


@functools.cache
def _segsum():
    return pl.kernel(
        _segsum_body,
        mesh=_mesh(),
        out_type=jax.ShapeDtypeStruct((E_PAD, 2 * D_SBF), jnp.float32),
        scratch_types=[
            pltpu.VMEM((3 * PCH, D_SBF), jnp.float32),
            pltpu.VMEM((6, 128), jnp.int32),
            pltpu.VMEM((MAIN_PER_W + CHUNK,), jnp.int32),
            pltpu.VMEM_SHARED((ACC_ROWS, D_SBF), jnp.float32),
            pltpu.SemaphoreType.DMA,
            pltpu.SemaphoreType.DMA,
        ],
        compiler_params=pltpu.CompilerParams(use_tc_tiling_on_sc=False),
    )


def _gather_body(x_hbm, idxs_hbm, out_hbm, idx_v, rows_v, sem):
    c = lax.axis_index("c")
    s = lax.axis_index("s")
    wid = s * 2 + c
    base = wid * PER_TILE
    sbases = [jnp.minimum(base + j * GB, E - GB) for j in range(N_GB)]
    for j in range(N_GB):
        pltpu.sync_copy(idxs_hbm.at[pl.ds(sbases[j], GB)], idx_v.at[j])
    for half in range(2):
        descs = []
        for j in range(N_GB // 2):
            g = half * (N_GB // 2) + j
            descs.append(pltpu.async_copy(x_hbm.at[idx_v.at[g]],
                                          rows_v.at[pl.ds(j * GB, GB)], sem))
        for d in descs:
            d.wait()
        for j in range(N_GB // 2):
            g = half * (N_GB // 2) + j
            pltpu.sync_copy(rows_v.at[pl.ds(j * GB, GB)],
                            out_hbm.at[pl.ds(sbases[g], GB)])


@functools.cache
def _gather():
    return pl.kernel(
        _gather_body,
        mesh=_mesh(),
        out_type=jax.ShapeDtypeStruct((E_PAD, 2 * D_QUAD), jnp.float32),
        scratch_types=[
            pltpu.VMEM((N_GB, GB), jnp.int32),
            pltpu.VMEM((PER_TILE // 2, 2 * D_QUAD), jnp.float32),
            pltpu.SemaphoreType.DMA,
        ],
    )


def _h_prep_body(m_ref, wd_ref, wb_ref, wst_ref, wts_ref, h_ref, wbt_ref,
                 fold_ref, wstp_ref, wtsp_ref):
    h = jnp.dot(m_ref[...], wd_ref[...], preferred_element_type=jnp.float32)
    h_ref[...] = h * jax.nn.sigmoid(h)

    @pl.when(pl.program_id(0) == 0)
    def _():
        z = jnp.zeros((D_SBF, D_QUAD * D_QUAD), jnp.float32)
        w3 = wb_ref[...]
        wbt = jnp.transpose(w3, (0, 2, 1)).reshape(D_SBF, D_QUAD * D_QUAD)
        wbt_ref[...] = jnp.concatenate([wbt, z], axis=0)
        r = lax.broadcasted_iota(jnp.int32, (D_QUAD * D_QUAD, 2 * D_QUAD), 0)
        c = lax.broadcasted_iota(jnp.int32, (D_QUAD * D_QUAD, 2 * D_QUAD), 1)
        fold_ref[...] = jnp.where(r // D_QUAD == c, 1.0, 0.0)
        zw = jnp.zeros((D_QUAD, D_EDGE), jnp.float32)
        wstp_ref[...] = jnp.concatenate([wst_ref[...], zw], axis=0)
        wtsp_ref[...] = jnp.concatenate([wts_ref[...], zw], axis=0)


def _bilinear_body(h_ref, s_ref, wbt_ref, fold_ref, x_ref):
    h = h_ref[...]
    t = jnp.dot(s_ref[...], wbt_ref[...], preferred_element_type=jnp.float32)
    hrep = pltpu.repeat(h, D_QUAD, axis=1)
    x_ref[...] = jnp.dot(t * hrep, fold_ref[...],
                         preferred_element_type=jnp.float32)


def _up_body(x_ref, xs_ref, ws_ref, wt_ref, o_ref):
    a = jnp.dot(x_ref[...], ws_ref[...], preferred_element_type=jnp.float32)
    b = jnp.dot(xs_ref[...], wt_ref[...], preferred_element_type=jnp.float32)
    o_ref[...] = (a * jax.nn.sigmoid(a) + b * jax.nn.sigmoid(b)) * INV_SQRT2


def kernel(m_st, sbf, idx_swap, edge_nb_idx, edge_nb_ragged_idx,
           W_down, W_bilinear, W_up_st, W_up_ts):
    del edge_nb_ragged_idx
    zeros = jnp.zeros((PER_TILE_Q, D_SBF), jnp.float32)

    h_pad, wbt, fold, wstp, wtsp = pl.pallas_call(
        _h_prep_body,
        grid=(GRID,),
        in_specs=[
            pl.BlockSpec((BLK, D_EDGE), lambda i: (i, 0)),
            pl.BlockSpec((D_EDGE, D_QUAD), lambda i: (0, 0)),
            pl.BlockSpec((D_SBF, D_QUAD, D_QUAD), lambda i: (0, 0, 0)),
            pl.BlockSpec((D_QUAD, D_EDGE), lambda i: (0, 0)),
            pl.BlockSpec((D_QUAD, D_EDGE), lambda i: (0, 0)),
        ],
        out_specs=(
            pl.BlockSpec((BLK, D_QUAD), lambda i: (i, 0)),
            pl.BlockSpec((2 * D_SBF, D_QUAD * D_QUAD), lambda i: (0, 0)),
            pl.BlockSpec((D_QUAD * D_QUAD, 2 * D_QUAD), lambda i: (0, 0)),
            pl.BlockSpec((2 * D_QUAD, D_EDGE), lambda i: (0, 0)),
            pl.BlockSpec((2 * D_QUAD, D_EDGE), lambda i: (0, 0)),
        ),
        out_shape=(
            jax.ShapeDtypeStruct((E_PAD, D_QUAD), jnp.float32),
            jax.ShapeDtypeStruct((2 * D_SBF, D_QUAD * D_QUAD), jnp.float32),
            jax.ShapeDtypeStruct((D_QUAD * D_QUAD, 2 * D_QUAD), jnp.float32),
            jax.ShapeDtypeStruct((2 * D_QUAD, D_EDGE), jnp.float32),
            jax.ShapeDtypeStruct((2 * D_QUAD, D_EDGE), jnp.float32),
        ),
    )(m_st, W_down, W_bilinear, W_up_st, W_up_ts)

    s_pad = _segsum()(sbf, edge_nb_idx.astype(jnp.int32), zeros)

    x_pad = pl.pallas_call(
        _bilinear_body,
        grid=(GRID,),
        in_specs=[
            pl.BlockSpec((BLK, D_QUAD), lambda i: (i, 0)),
            pl.BlockSpec((BLK, 2 * D_SBF), lambda i: (i, 0)),
            pl.BlockSpec((2 * D_SBF, D_QUAD * D_QUAD), lambda i: (0, 0)),
            pl.BlockSpec((D_QUAD * D_QUAD, 2 * D_QUAD), lambda i: (0, 0)),
        ],
        out_specs=pl.BlockSpec((BLK, 2 * D_QUAD), lambda i: (i, 0)),
        out_shape=jax.ShapeDtypeStruct((E_PAD, 2 * D_QUAD), jnp.float32),
    )(h_pad, s_pad, wbt, fold)

    x_sw = _gather()(x_pad, idx_swap.astype(jnp.int32))

    out = pl.pallas_call(
        _up_body,
        grid=(GRID,),
        in_specs=[
            pl.BlockSpec((BLK, 2 * D_QUAD), lambda i: (i, 0)),
            pl.BlockSpec((BLK, 2 * D_QUAD), lambda i: (i, 0)),
            pl.BlockSpec((2 * D_QUAD, D_EDGE), lambda i: (0, 0)),
            pl.BlockSpec((2 * D_QUAD, D_EDGE), lambda i: (0, 0)),
        ],
        out_specs=pl.BlockSpec((BLK, D_EDGE), lambda i: (i, 0)),
        out_shape=jax.ShapeDtypeStruct((E, D_EDGE), jnp.float32),
    )(x_pad, x_sw, wstp, wtsp)

    return out


# speedup vs baseline: 1.0014x; 1.0013x over previous
import functools

import jax
import jax.numpy as jnp
from jax import lax
from jax.experimental import pallas as pl
from jax.experimental.pallas import tpu as pltpu
from jax.experimental.pallas import tpu_sc as plsc

E = 50000
E_NB = 400000
D_EDGE = 512
D_SBF = 64
D_QUAD = 64

HALF = 25088
E_PAD = 2 * HALF
QUARTER = HALF // 2
TRASH = QUARTER
ACC_ROWS = QUARTER + 8
PER_TILE = HALF // 16
PER_TILE_Q = QUARTER // 16

CHUNK = 512
PCH = 256
MAIN_PER_W = 24576
N_MAIN = MAIN_PER_W // PCH
EXTRA_BASE = 16 * MAIN_PER_W
TAIL2_BASE = EXTRA_BASE + 13 * CHUNK

GB = 112
N_GB = PER_TILE // GB

BLK = 1024
GRID = E_PAD // BLK
INV_SQRT2 = 0.7071067811865476

@functools.cache
def _mesh():
    return plsc.VectorSubcoreMesh(core_axis_name="c", subcore_axis_name="s")


def _segsum_body(sbf_hbm, idx_hbm, zeros_hbm, out_hbm, rows_v, idx_sc,
                 idx_all, acc_sh, sem, sem2):
    c = lax.axis_index("c")
    s = lax.axis_index("s")
    mb = s * MAIN_PER_W

    pltpu.sync_copy(idx_hbm.at[pl.ds(mb, MAIN_PER_W)],
                    idx_all.at[pl.ds(0, MAIN_PER_W)])

    @pl.when(s < 13)
    def _():
        pltpu.sync_copy(idx_hbm.at[pl.ds(EXTRA_BASE + s * CHUNK, CHUNK)],
                        idx_all.at[pl.ds(MAIN_PER_W, CHUNK)])

    @pl.when(s == 13)
    def _():
        pltpu.sync_copy(idx_hbm.at[pl.ds(TAIL2_BASE, 128)],
                        idx_all.at[pl.ds(MAIN_PER_W, 128)])

    for p in range(2):
        lo = c * HALF + p * QUARTER
        hi = lo + QUARTER

        pltpu.sync_copy(zeros_hbm, acc_sh.at[pl.ds(s * PER_TILE_Q, PER_TILE_Q)])

        @pl.when(s == 0)
        def _():
            pltpu.sync_copy(zeros_hbm.at[pl.ds(0, 8)],
                            acc_sh.at[pl.ds(QUARTER, 8)])

        plsc.subcore_barrier()

        n_lo = jnp.int32(-1)
        n_hi = jnp.int32(-2)
        for n in range(N_MAIN):
            f = idx_all[pl.ds(n * PCH, 16)][0]
            l = idx_all[pl.ds(n * PCH + PCH - 16, 16)][15]
            inr = (f < hi) & (l >= lo)
            n_lo = jnp.where(inr & (n_lo < 0), n, n_lo)
            n_hi = jnp.where(inr, n, n_hi)

        def transform(base, nvec, slot, lo=lo, hi=hi):
            for v in range(nvec):
                vec = idx_all[pl.ds(base + v * 16, 16)]
                inr = (vec >= lo) & (vec < hi)
                idx_sc[slot * 2 + v // 8, pl.ds((v % 8) * 16, 16)] = jnp.where(
                    inr, vec - lo, TRASH)

        def issue(k):
            pltpu.async_copy(sbf_hbm.at[pl.ds(mb + k * PCH, PCH)],
                             rows_v.at[pl.ds((k % 3) * PCH, PCH)], sem)

        def drain_scatter():
            pltpu.make_async_copy(sbf_hbm.at[pl.ds(0, PCH)],
                                  rows_v.at[pl.ds(0, PCH)], sem2).wait()

        @pl.when(n_hi >= n_lo)
        def _():
            issue(n_lo)

            @pl.when(n_lo + 1 <= n_hi)
            def _():
                issue(n_lo + 1)

            def body(n, carry):
                @pl.when(n >= n_lo + 2)
                def _():
                    drain_scatter()

                @pl.when(n + 2 <= n_hi)
                def _():
                    issue(n + 2)

                transform(n * PCH, PCH // 16, n % 3)
                pltpu.make_async_copy(
                    sbf_hbm.at[pl.ds(mb + n * PCH, PCH)],
                    rows_v.at[pl.ds((n % 3) * PCH, PCH)], sem).wait()
                for j in range(PCH // 128):
                    pltpu.async_copy(
                        rows_v.at[pl.ds((n % 3) * PCH + j * 128, 128)],
                        acc_sh.at[idx_sc.at[(n % 3) * 2 + j]], sem2, add=True)
                return carry

            lax.fori_loop(n_lo, n_hi + 1, body, 0)
            drain_scatter()

            @pl.when(n_hi > n_lo)
            def _():
                drain_scatter()

        def extra(base, nrows, lo=lo, hi=hi):
            nsub = nrows // 128
            first = idx_all[pl.ds(MAIN_PER_W, 16)][0]
            last = idx_all[pl.ds(MAIN_PER_W + nrows - 16, 16)][15]

            @pl.when((first < hi) & (last >= lo))
            def _():
                transform(MAIN_PER_W, nrows // 16, 0)
                pltpu.sync_copy(sbf_hbm.at[pl.ds(base, nrows)],
                                rows_v.at[pl.ds(0, nrows)])
                for j in range(nsub):
                    pltpu.sync_copy(rows_v.at[pl.ds(j * 128, 128)],
                                    acc_sh.at[idx_sc.at[j]], add=True)

        @pl.when(s < 13)
        def _():
            extra(EXTRA_BASE + s * CHUNK, CHUNK)

        @pl.when(s == 13)
        def _():
            extra(TAIL2_BASE, 128)

        plsc.subcore_barrier()
        pltpu.sync_copy(acc_sh.at[pl.ds(s * PER_TILE_Q, PER_TILE_Q)],
                        out_hbm.at[pl.ds(lo + s * PER_TILE_Q, PER_TILE_Q),
                                   pl.ds(0, D_SBF)])
        pltpu.sync_copy(acc_sh.at[pl.ds(s * PER_TILE_Q, PER_TILE_Q)],
                        out_hbm.at[pl.ds(lo + s * PER_TILE_Q, PER_TILE_Q),
                                   pl.ds(D_SBF, D_SBF)])
        plsc.subcore_barrier()


@functools.cache
def _segsum():
    return pl.kernel(
        _segsum_body,
        mesh=_mesh(),
        out_type=jax.ShapeDtypeStruct((E_PAD, 2 * D_SBF), jnp.float32),
        scratch_types=[
            pltpu.VMEM((3 * PCH, D_SBF), jnp.float32),
            pltpu.VMEM((6, 128), jnp.int32),
            pltpu.VMEM((MAIN_PER_W + CHUNK,), jnp.int32),
            pltpu.VMEM_SHARED((ACC_ROWS, D_SBF), jnp.float32),
            pltpu.SemaphoreType.DMA,
            pltpu.SemaphoreType.DMA,
        ],
        compiler_params=pltpu.CompilerParams(use_tc_tiling_on_sc=False),
    )


def _gather_body(x_hbm, idxs_hbm, out_hbm, idx_v, rows_v, sem):
    c = lax.axis_index("c")
    s = lax.axis_index("s")
    wid = s * 2 + c
    base = wid * PER_TILE
    sbases = [jnp.minimum(base + j * GB, E - GB) for j in range(N_GB)]
    for j in range(N_GB):
        pltpu.sync_copy(idxs_hbm.at[pl.ds(sbases[j], GB)], idx_v.at[j])
    for half in range(2):
        descs = []
        for j in range(N_GB // 2):
            g = half * (N_GB // 2) + j
            descs.append(pltpu.async_copy(x_hbm.at[idx_v.at[g]],
                                          rows_v.at[pl.ds(j * GB, GB)], sem))
        for d in descs:
            d.wait()
        for j in range(N_GB // 2):
            g = half * (N_GB // 2) + j
            pltpu.sync_copy(rows_v.at[pl.ds(j * GB, GB)],
                            out_hbm.at[pl.ds(sbases[g], GB)])


@functools.cache
def _gather():
    return pl.kernel(
        _gather_body,
        mesh=_mesh(),
        out_type=jax.ShapeDtypeStruct((E_PAD, 2 * D_QUAD), jnp.float32),
        scratch_types=[
            pltpu.VMEM((N_GB, GB), jnp.int32),
            pltpu.VMEM((PER_TILE // 2, 2 * D_QUAD), jnp.float32),
            pltpu.SemaphoreType.DMA,
        ],
    )


def _h_prep_body(m_ref, wd_ref, wb_ref, wst_ref, wts_ref, h_ref, wbt_ref,
                 fold_ref, wstp_ref, wtsp_ref):
    h = jnp.dot(m_ref[...], wd_ref[...], preferred_element_type=jnp.float32)
    h_ref[...] = h * jax.nn.sigmoid(h)

    @pl.when(pl.program_id(0) == 0)
    def _():
        z = jnp.zeros((D_SBF, D_QUAD * D_QUAD), jnp.float32)
        w3 = wb_ref[...]
        wbt = jnp.transpose(w3, (0, 2, 1)).reshape(D_SBF, D_QUAD * D_QUAD)
        wbt_ref[...] = jnp.concatenate([wbt, z], axis=0)
        r = lax.broadcasted_iota(jnp.int32, (D_QUAD * D_QUAD, 2 * D_QUAD), 0)
        c = lax.broadcasted_iota(jnp.int32, (D_QUAD * D_QUAD, 2 * D_QUAD), 1)
        fold_ref[...] = jnp.where(r // D_QUAD == c, 1.0, 0.0)
        zw = jnp.zeros((D_QUAD, D_EDGE), jnp.float32)
        wstp_ref[...] = jnp.concatenate([wst_ref[...], zw], axis=0)
        wtsp_ref[...] = jnp.concatenate([wts_ref[...], zw], axis=0)


def _bilinear_body(h_ref, s_ref, wbt_ref, fold_ref, x_ref):
    h = h_ref[...]
    t = jnp.dot(s_ref[...], wbt_ref[...], preferred_element_type=jnp.float32)
    hrep = pltpu.repeat(h, D_QUAD, axis=1)
    x_ref[...] = jnp.dot(t * hrep, fold_ref[...],
                         preferred_element_type=jnp.float32)


def _up_body(x_ref, xs_ref, ws_ref, wt_ref, o_ref):
    a = jnp.dot(x_ref[...], ws_ref[...], preferred_element_type=jnp.float32)
    b = jnp.dot(xs_ref[...], wt_ref[...], preferred_element_type=jnp.float32)
    o_ref[...] = (a * jax.nn.sigmoid(a) + b * jax.nn.sigmoid(b)) * INV_SQRT2


def kernel(m_st, sbf, idx_swap, edge_nb_idx, edge_nb_ragged_idx,
           W_down, W_bilinear, W_up_st, W_up_ts):
    del edge_nb_ragged_idx
    zeros = jnp.zeros((PER_TILE_Q, D_SBF), jnp.float32)

    h_pad, wbt, fold, wstp, wtsp = pl.pallas_call(
        _h_prep_body,
        grid=(GRID,),
        in_specs=[
            pl.BlockSpec((BLK, D_EDGE), lambda i: (i, 0)),
            pl.BlockSpec((D_EDGE, D_QUAD), lambda i: (0, 0)),
            pl.BlockSpec((D_SBF, D_QUAD, D_QUAD), lambda i: (0, 0, 0)),
            pl.BlockSpec((D_QUAD, D_EDGE), lambda i: (0, 0)),
            pl.BlockSpec((D_QUAD, D_EDGE), lambda i: (0, 0)),
        ],
        out_specs=(
            pl.BlockSpec((BLK, D_QUAD), lambda i: (i, 0)),
            pl.BlockSpec((2 * D_SBF, D_QUAD * D_QUAD), lambda i: (0, 0)),
            pl.BlockSpec((D_QUAD * D_QUAD, 2 * D_QUAD), lambda i: (0, 0)),
            pl.BlockSpec((2 * D_QUAD, D_EDGE), lambda i: (0, 0)),
            pl.BlockSpec((2 * D_QUAD, D_EDGE), lambda i: (0, 0)),
        ),
        out_shape=(
            jax.ShapeDtypeStruct((E_PAD, D_QUAD), jnp.float32),
            jax.ShapeDtypeStruct((2 * D_SBF, D_QUAD * D_QUAD), jnp.float32),
            jax.ShapeDtypeStruct((D_QUAD * D_QUAD, 2 * D_QUAD), jnp.float32),
            jax.ShapeDtypeStruct((2 * D_QUAD, D_EDGE), jnp.float32),
            jax.ShapeDtypeStruct((2 * D_QUAD, D_EDGE), jnp.float32),
        ),
    )(m_st, W_down, W_bilinear, W_up_st, W_up_ts)

    s_pad = _segsum()(sbf, edge_nb_idx.astype(jnp.int32), zeros)

    x_pad = pl.pallas_call(
        _bilinear_body,
        grid=(GRID,),
        in_specs=[
            pl.BlockSpec((BLK, D_QUAD), lambda i: (i, 0)),
            pl.BlockSpec((BLK, 2 * D_SBF), lambda i: (i, 0)),
            pl.BlockSpec((2 * D_SBF, D_QUAD * D_QUAD), lambda i: (0, 0)),
            pl.BlockSpec((D_QUAD * D_QUAD, 2 * D_QUAD), lambda i: (0, 0)),
        ],
        out_specs=pl.BlockSpec((BLK, 2 * D_QUAD), lambda i: (i, 0)),
        out_shape=jax.ShapeDtypeStruct((E_PAD, 2 * D_QUAD), jnp.float32),
    )(h_pad, s_pad, wbt, fold)

    x_sw = _gather()(x_pad, idx_swap.astype(jnp.int32))

    out = pl.pallas_call(
        _up_body,
        grid=(GRID,),
        in_specs=[
            pl.BlockSpec((BLK, 2 * D_QUAD), lambda i: (i, 0)),
            pl.BlockSpec((BLK, 2 * D_QUAD), lambda i: (i, 0)),
            pl.BlockSpec((2 * D_QUAD, D_EDGE), lambda i: (0, 0)),
            pl.BlockSpec((2 * D_QUAD, D_EDGE), lambda i: (0, 0)),
        ],
        out_specs=pl.BlockSpec((BLK, D_EDGE), lambda i: (i, 0)),
        out_shape=jax.ShapeDtypeStruct((E, D_EDGE), jnp.float32),
    )(x_pad, x_sw, wstp, wtsp)

    return out
